# baseline (speedup 1.0000x reference)
import jax
import jax.numpy as jnp
from jax import lax
from jax.experimental import pallas as pl
from jax.experimental.pallas import tpu as pltpu
from jax.experimental.pallas import tpu_sc as plsc

N_COARSE = 262144
C = 32
N_FINE = 8 * N_COARSE

NUM_CORES = 2
NUM_SUBCORES = 16
NW = NUM_CORES * NUM_SUBCORES
ROWS_PER_W = N_COARSE // NW
CHUNK = 128
FINE_PER_CHUNK = 8 * CHUNK
N_CHUNKS = ROWS_PER_W // CHUNK
L = 16
DWORDS = CHUNK * C
OWORDS = FINE_PER_CHUNK * C


def _unpool_body(data_hbm, mask_hbm, out_hbm,
                 dbuf0, dbuf1, mbuf0, mbuf1, wbuf0, wbuf1, obuf0, obuf1,
                 isem0, isem1, osem0, osem1):
    wid = lax.axis_index("s") * NUM_CORES + lax.axis_index("c")
    w_base = wid * ROWS_PER_W

    dbufs = (dbuf0, dbuf1)
    mbufs = (mbuf0, mbuf1)
    wbufs = (wbuf0, wbuf1)
    obufs = (obuf0, obuf1)
    isems = (isem0, isem1)
    osems = (osem0, osem1)

    lanes = lax.iota(jnp.int32, L)
    lanes_o = lanes * (8 * C)
    zvec = jnp.zeros((L,), jnp.float32)

    def start_in(chunk, p):
        base = w_base + chunk * CHUNK
        pltpu.make_async_copy(
            data_hbm.at[pl.ds(base * C, DWORDS)], dbufs[p], isems[p]).start()
        pltpu.make_async_copy(
            mask_hbm.at[pl.ds(base, CHUNK)], mbufs[p], isems[p]).start()

    def wait_in(p):
        pltpu.make_async_copy(
            data_hbm.at[pl.ds(0, DWORDS)], dbufs[p], isems[p]).wait()
        pltpu.make_async_copy(
            mask_hbm.at[pl.ds(0, CHUNK)], mbufs[p], isems[p]).wait()

    def zinit(j, _):
        obuf0[pl.ds(j * L, L)] = zvec
        obuf1[pl.ds(j * L, L)] = zvec
        return 0

    lax.fori_loop(0, OWORDS // L, zinit, 0)

    start_in(0, 0)
    start_in(1, 1)

    def step(g2, _):
        for p in range(2):
            chunk = g2 * 2 + p
            base = w_base + chunk * CHUNK
            dbuf, mbuf, wbuf, obuf = dbufs[p], mbufs[p], wbufs[p], obufs[p]

            wait_in(p)

            @pl.when(g2 > 0)
            def _():
                pltpu.make_async_copy(
                    obuf, out_hbm.at[pl.ds(0, OWORDS)], osems[p]).wait()

                def erase_body(b, _):
                    v = wbuf[pl.ds(b * L, L)]
                    for k in range(L):
                        pw = v[k]
                        obuf[pl.ds(pw, L)] = zvec
                        obuf[pl.ds(pw + L, L)] = zvec
                    return 0

                lax.fori_loop(0, CHUNK // L, erase_body, 0)

            for b in range(CHUNK // L):
                m = mbuf[pl.ds(b * L, L)]
                wbuf[pl.ds(b * L, L)] = (b * (L * 8 * C)) + lanes_o + m * C

            def scat_body(b, _):
                v = wbuf[pl.ds(b * L, L)]
                for k in range(L):
                    wi = v[k]
                    i = b * L + k
                    obuf[pl.ds(wi, L)] = dbuf[pl.ds(i * C, L)]
                    obuf[pl.ds(wi + L, L)] = dbuf[pl.ds(i * C + L, L)]
                return 0

            lax.fori_loop(0, CHUNK // L, scat_body, 0)

            pltpu.make_async_copy(
                obuf, out_hbm.at[pl.ds(base * 8 * C, OWORDS)],
                osems[p]).start()

            @pl.when(g2 < (N_CHUNKS // 2) - 1)
            def _():
                start_in(chunk + 2, p)
        return 0

    lax.fori_loop(0, N_CHUNKS // 2, step, 0)

    for p in range(2):
        pltpu.make_async_copy(
            obufs[p], out_hbm.at[pl.ds(0, OWORDS)], osems[p]).wait()


@jax.jit
def _unpool(data, mask):
    f = pl.kernel(
        _unpool_body,
        out_type=jax.ShapeDtypeStruct((N_FINE * C,), jnp.float32),
        mesh=plsc.VectorSubcoreMesh(core_axis_name="c", subcore_axis_name="s"),
        scratch_types=[
            pltpu.VMEM((DWORDS,), jnp.float32),
            pltpu.VMEM((DWORDS,), jnp.float32),
            pltpu.VMEM((CHUNK,), jnp.int32),
            pltpu.VMEM((CHUNK,), jnp.int32),
            pltpu.VMEM((CHUNK,), jnp.int32),
            pltpu.VMEM((CHUNK,), jnp.int32),
            pltpu.VMEM((OWORDS,), jnp.float32),
            pltpu.VMEM((OWORDS,), jnp.float32),
            pltpu.SemaphoreType.DMA,
            pltpu.SemaphoreType.DMA,
            pltpu.SemaphoreType.DMA,
            pltpu.SemaphoreType.DMA,
        ],
        compiler_params=pltpu.CompilerParams(
            needs_layout_passes=False, use_tc_tiling_on_sc=False),
    )
    return f(data.reshape(N_COARSE * C), mask)


def kernel(data, mask, octree):
    del octree
    return _unpool(data, mask).reshape(N_FINE, C)


# speedup vs baseline: 1.0047x; 1.0047x over previous
import jax
import jax.numpy as jnp
from jax import lax
from jax.experimental import pallas as pl
from jax.experimental.pallas import tpu as pltpu
from jax.experimental.pallas import tpu_sc as plsc

N_COARSE = 262144
C = 32
N_FINE = 8 * N_COARSE

NUM_CORES = 2
NUM_SUBCORES = 16
NW = NUM_CORES * NUM_SUBCORES
ROWS_PER_W = N_COARSE // NW
CHUNK = 128
FINE_PER_CHUNK = 8 * CHUNK
N_CHUNKS = ROWS_PER_W // CHUNK
L = 16
DWORDS = CHUNK * C
OWORDS = FINE_PER_CHUNK * C


def _unpool_body(data_hbm, mask_hbm, out_hbm,
                 dbuf0, dbuf1, mbuf0, mbuf1, wbuf0, wbuf1, obuf0, obuf1,
                 isem0, isem1, osem0, osem1):
    wid = lax.axis_index("s") * NUM_CORES + lax.axis_index("c")
    w_base = wid * ROWS_PER_W

    dbufs = (dbuf0, dbuf1)
    mbufs = (mbuf0, mbuf1)
    wbufs = (wbuf0, wbuf1)
    obufs = (obuf0, obuf1)
    isems = (isem0, isem1)
    osems = (osem0, osem1)

    lanes = lax.iota(jnp.int32, L)
    lanes_o = lanes * (8 * C)
    zvec = jnp.zeros((L,), jnp.float32)

    def start_in(chunk, p):
        base = w_base + chunk * CHUNK
        pltpu.make_async_copy(
            data_hbm.at[pl.ds(base * C, DWORDS)], dbufs[p], isems[p]).start()
        pltpu.make_async_copy(
            mask_hbm.at[pl.ds(base, CHUNK)], mbufs[p], isems[p]).start()

    def wait_in(p):
        pltpu.make_async_copy(
            data_hbm.at[pl.ds(0, DWORDS)], dbufs[p], isems[p]).wait()
        pltpu.make_async_copy(
            mask_hbm.at[pl.ds(0, CHUNK)], mbufs[p], isems[p]).wait()

    def zinit(j, _):
        obuf0[pl.ds(j * L, L)] = zvec
        obuf1[pl.ds(j * L, L)] = zvec
        return 0

    lax.fori_loop(0, OWORDS // L, zinit, 0)

    start_in(0, 0)
    start_in(1, 1)

    def step(g2, _):
        for p in range(2):
            chunk = g2 * 2 + p
            base = w_base + chunk * CHUNK
            dbuf, mbuf, wbuf, obuf = dbufs[p], mbufs[p], wbufs[p], obufs[p]

            wait_in(p)

            @pl.when(g2 > 0)
            def _():
                pltpu.make_async_copy(
                    obuf, out_hbm.at[pl.ds(0, OWORDS)], osems[p]).wait()

                def erase_body(b, _):
                    v = wbuf[pl.ds(b * L, L)]
                    for k in range(L):
                        pw = v[k]
                        obuf[pl.ds(pw, L)] = zvec
                        obuf[pl.ds(pw + L, L)] = zvec
                    return 0

                lax.fori_loop(0, CHUNK // L, erase_body, 0)

            for b in range(CHUNK // L):
                m = mbuf[pl.ds(b * L, L)]
                wbuf[pl.ds(b * L, L)] = (b * (L * 8 * C)) + lanes_o + m * C

            def scat_body(b, _):
                v = wbuf[pl.ds(b * L, L)]
                for k in range(L):
                    wi = v[k]
                    i = b * L + k
                    obuf[pl.ds(wi, L)] = dbuf[pl.ds(i * C, L)]
                    obuf[pl.ds(wi + L, L)] = dbuf[pl.ds(i * C + L, L)]
                return 0

            lax.fori_loop(0, CHUNK // L, scat_body, 0)

            plsc.subcore_barrier()

            pltpu.make_async_copy(
                obuf, out_hbm.at[pl.ds(base * 8 * C, OWORDS)],
                osems[p]).start()

            @pl.when(g2 < (N_CHUNKS // 2) - 1)
            def _():
                start_in(chunk + 2, p)
        return 0

    lax.fori_loop(0, N_CHUNKS // 2, step, 0)

    for p in range(2):
        pltpu.make_async_copy(
            obufs[p], out_hbm.at[pl.ds(0, OWORDS)], osems[p]).wait()


@jax.jit
def _unpool(data, mask):
    f = pl.kernel(
        _unpool_body,
        out_type=jax.ShapeDtypeStruct((N_FINE * C,), jnp.float32),
        mesh=plsc.VectorSubcoreMesh(core_axis_name="c", subcore_axis_name="s"),
        scratch_types=[
            pltpu.VMEM((DWORDS,), jnp.float32),
            pltpu.VMEM((DWORDS,), jnp.float32),
            pltpu.VMEM((CHUNK,), jnp.int32),
            pltpu.VMEM((CHUNK,), jnp.int32),
            pltpu.VMEM((CHUNK,), jnp.int32),
            pltpu.VMEM((CHUNK,), jnp.int32),
            pltpu.VMEM((OWORDS,), jnp.float32),
            pltpu.VMEM((OWORDS,), jnp.float32),
            pltpu.SemaphoreType.DMA,
            pltpu.SemaphoreType.DMA,
            pltpu.SemaphoreType.DMA,
            pltpu.SemaphoreType.DMA,
        ],
        compiler_params=pltpu.CompilerParams(
            needs_layout_passes=False, use_tc_tiling_on_sc=False),
    )
    return f(data.reshape(N_COARSE * C), mask)


def kernel(data, mask, octree):
    del octree
    return _unpool(data, mask).reshape(N_FINE, C)
